# baseline (speedup 1.0000x reference)
import jax
import jax.numpy as jnp
from jax.experimental import pallas as pl
from jax.experimental.pallas import tpu as pltpu

N_NODES = 100000
D = 512
H = 256
NSEG = 64
TILE = 10000
NTILES = N_NODES // TILE


def _body(x_ref, w1_ref, b1_ref, w2_ref, b2_ref, seg_ref, out_ref,
          acc_ref, d_ref, cnt_ref):
    i = pl.program_id(0)

    @pl.when(i == 0)
    def _init():
        acc_ref[...] = jnp.zeros_like(acc_ref)
        d_ref[...] = jnp.zeros_like(d_ref)
        cnt_ref[...] = jnp.zeros_like(cnt_ref)

    xb16 = x_ref[...].astype(jnp.bfloat16)
    w1 = w1_ref[...].astype(jnp.bfloat16)
    h = jnp.dot(xb16, w1, preferred_element_type=jnp.float32)
    h = jnp.tanh(h + b1_ref[...])
    w2 = w2_ref[...]
    s = jnp.sum(h * w2, axis=1, keepdims=True) + b2_ref[0, 0]
    ex = jnp.exp(s)

    seg = seg_ref[0]
    ids = jax.lax.broadcasted_iota(jnp.int32, (TILE, NSEG), 1)
    mask = seg.reshape(TILE, 1) == ids

    q = jnp.where(mask, ex, 0.0)
    d_ref[...] = d_ref[...] + jnp.sum(q, axis=0, keepdims=True)
    cnt_ref[...] = cnt_ref[...] + jnp.sum(
        jnp.where(mask, 1.0, 0.0), axis=0, keepdims=True)

    contrib = jax.lax.dot_general(
        q.astype(jnp.bfloat16), xb16,
        dimension_numbers=(((0,), (0,)), ((), ())),
        preferred_element_type=jnp.float32)
    acc_ref[...] = acc_ref[...] + contrib

    @pl.when(i == NTILES - 1)
    def _fini():
        denom = d_ref[...].reshape(NSEG, 1) * cnt_ref[...].reshape(NSEG, 1)
        good = cnt_ref[...].reshape(NSEG, 1) > 0.0
        out_ref[...] = jnp.where(good, acc_ref[...] / jnp.where(good, denom, 1.0),
                                 0.0)


@jax.jit
def kernel(x, W1, b1, W2, b2, batch):
    seg = batch.astype(jnp.int32).reshape(NTILES, 1, TILE)
    b1r = b1.reshape(1, H).astype(jnp.float32)
    w2r = W2.reshape(1, H).astype(jnp.float32)
    c = jnp.sum(jnp.abs(w2r)) + jnp.abs(b2[0])
    b2r = (b2.astype(jnp.float32) - c).reshape(1, 1)
    out = pl.pallas_call(
        _body,
        grid=(NTILES,),
        in_specs=[
            pl.BlockSpec((TILE, D), lambda i: (i, 0)),
            pl.BlockSpec((D, H), lambda i: (0, 0)),
            pl.BlockSpec((1, H), lambda i: (0, 0)),
            pl.BlockSpec((1, H), lambda i: (0, 0)),
            pl.BlockSpec((1, 1), lambda i: (0, 0)),
            pl.BlockSpec((1, 1, TILE), lambda i: (i, 0, 0)),
        ],
        out_specs=pl.BlockSpec((NSEG, D), lambda i: (0, 0)),
        out_shape=jax.ShapeDtypeStruct((NSEG, D), jnp.float32),
        scratch_shapes=[
            pltpu.VMEM((NSEG, D), jnp.float32),
            pltpu.VMEM((1, NSEG), jnp.float32),
            pltpu.VMEM((1, NSEG), jnp.float32),
        ],
        compiler_params=pltpu.CompilerParams(
            dimension_semantics=("arbitrary",)),
    )(x, W1, b1r, w2r, b2r, seg)
    return out


# speedup vs baseline: 1.0828x; 1.0828x over previous
import jax
import jax.numpy as jnp
from jax.experimental import pallas as pl
from jax.experimental.pallas import tpu as pltpu

N_NODES = 100000
D = 512
H = 256
NSEG = 64
TILE = 5000
NTILES = N_NODES // TILE


def _body(x_ref, w1_ref, b1_ref, w2_ref, b2_ref, seg_ref, out_ref,
          acc_ref, d_ref, cnt_ref):
    i = pl.program_id(0)

    @pl.when(i == 0)
    def _init():
        acc_ref[...] = jnp.zeros_like(acc_ref)
        d_ref[...] = jnp.zeros_like(d_ref)
        cnt_ref[...] = jnp.zeros_like(cnt_ref)

    xb16 = x_ref[...].astype(jnp.bfloat16)
    w1 = w1_ref[...].astype(jnp.bfloat16)
    hp = jnp.dot(xb16, w1, preferred_element_type=jnp.float32)
    h = jnp.tanh(hp + b1_ref[...])
    w2 = w2_ref[...]
    s = jnp.sum(h * w2, axis=1, keepdims=True) + b2_ref[0, 0]
    ex = jnp.exp(s.reshape(1, TILE))

    seg = seg_ref[0]
    ids = jax.lax.broadcasted_iota(jnp.int32, (NSEG, TILE), 0)
    mask = ids == seg

    q = jnp.where(mask, ex, 0.0)
    d_ref[...] = d_ref[...] + jnp.sum(q, axis=1, keepdims=True)
    cnt_ref[...] = cnt_ref[...] + jnp.sum(
        jnp.where(mask, 1.0, 0.0), axis=1, keepdims=True)

    contrib = jnp.dot(q.astype(jnp.bfloat16), xb16,
                      preferred_element_type=jnp.float32)
    acc_ref[...] = acc_ref[...] + contrib

    @pl.when(i == NTILES - 1)
    def _fini():
        denom = d_ref[...] * cnt_ref[...]
        good = cnt_ref[...] > 0.0
        out_ref[...] = jnp.where(good, acc_ref[...] / jnp.where(good, denom, 1.0),
                                 0.0)


@jax.jit
def kernel(x, W1, b1, W2, b2, batch):
    seg = batch.astype(jnp.int32).reshape(NTILES, 1, TILE)
    b1r = b1.reshape(1, H).astype(jnp.float32)
    w2r = W2.reshape(1, H).astype(jnp.float32)
    c = jnp.sum(jnp.abs(w2r)) + jnp.abs(b2[0])
    b2r = (b2.astype(jnp.float32) - c).reshape(1, 1)
    out = pl.pallas_call(
        _body,
        grid=(NTILES,),
        in_specs=[
            pl.BlockSpec((TILE, D), lambda i: (i, 0)),
            pl.BlockSpec((D, H), lambda i: (0, 0)),
            pl.BlockSpec((1, H), lambda i: (0, 0)),
            pl.BlockSpec((1, H), lambda i: (0, 0)),
            pl.BlockSpec((1, 1), lambda i: (0, 0)),
            pl.BlockSpec((1, 1, TILE), lambda i: (i, 0, 0)),
        ],
        out_specs=pl.BlockSpec((NSEG, D), lambda i: (0, 0)),
        out_shape=jax.ShapeDtypeStruct((NSEG, D), jnp.float32),
        scratch_shapes=[
            pltpu.VMEM((NSEG, D), jnp.float32),
            pltpu.VMEM((NSEG, 1), jnp.float32),
            pltpu.VMEM((NSEG, 1), jnp.float32),
        ],
        compiler_params=pltpu.CompilerParams(
            dimension_semantics=("arbitrary",)),
    )(x, W1, b1r, w2r, b2r, seg)
    return out


# speedup vs baseline: 1.1229x; 1.0371x over previous
import jax
import jax.numpy as jnp
from jax.experimental import pallas as pl
from jax.experimental.pallas import tpu as pltpu

N_NODES = 100000
D = 512
H = 256
NSEG = 64
TILE = 4000
NCHUNK = 2
CHUNK = TILE // NCHUNK
NTILES = N_NODES // TILE


def _body(x_ref, w1_ref, b1_ref, w2_ref, b2_ref, seg_ref, out_ref,
          acc_ref, d_ref, cnt_ref):
    i = pl.program_id(0)

    @pl.when(i == 0)
    def _init():
        acc_ref[...] = jnp.zeros_like(acc_ref)
        d_ref[...] = jnp.zeros_like(d_ref)
        cnt_ref[...] = jnp.zeros_like(cnt_ref)

    w1 = w1_ref[...].astype(jnp.bfloat16)
    w2 = w2_ref[...]
    b1 = b1_ref[...]
    b2 = b2_ref[0, 0]

    contribs, ds, cnts = [], [], []
    for k in range(NCHUNK):
        xb16 = x_ref[k * CHUNK:(k + 1) * CHUNK, :].astype(jnp.bfloat16)
        hp = jnp.dot(xb16, w1, preferred_element_type=jnp.float32)
        h = jnp.tanh(hp + b1)
        s = jnp.sum(h * w2, axis=1, keepdims=True) + b2
        ex = jnp.exp(s).reshape(1, CHUNK)

        seg = seg_ref[0, k].reshape(1, CHUNK)
        ids = jax.lax.broadcasted_iota(jnp.int32, (NSEG, CHUNK), 0)
        mask = ids == seg

        q = jnp.where(mask, ex, 0.0)
        ds.append(jnp.sum(q, axis=1, keepdims=True))
        cnts.append(jnp.sum(jnp.where(mask, 1.0, 0.0), axis=1, keepdims=True))
        contribs.append(jnp.dot(q.astype(jnp.bfloat16), xb16,
                                preferred_element_type=jnp.float32))

    acc_ref[...] = acc_ref[...] + sum(contribs)
    d_ref[...] = d_ref[...] + sum(ds)
    cnt_ref[...] = cnt_ref[...] + sum(cnts)

    @pl.when(i == NTILES - 1)
    def _fini():
        denom = d_ref[...] * cnt_ref[...]
        good = cnt_ref[...] > 0.0
        out_ref[...] = jnp.where(good, acc_ref[...] / jnp.where(good, denom, 1.0),
                                 0.0)


@jax.jit
def kernel(x, W1, b1, W2, b2, batch):
    seg = batch.astype(jnp.int32).reshape(NTILES, NCHUNK, CHUNK)
    b1r = b1.reshape(1, H).astype(jnp.float32)
    w2r = W2.reshape(1, H).astype(jnp.float32)
    c = jnp.sum(jnp.abs(w2r)) + jnp.abs(b2[0])
    b2r = (b2.astype(jnp.float32) - c).reshape(1, 1)
    out = pl.pallas_call(
        _body,
        grid=(NTILES,),
        in_specs=[
            pl.BlockSpec((TILE, D), lambda i: (i, 0)),
            pl.BlockSpec((D, H), lambda i: (0, 0)),
            pl.BlockSpec((1, H), lambda i: (0, 0)),
            pl.BlockSpec((1, H), lambda i: (0, 0)),
            pl.BlockSpec((1, 1), lambda i: (0, 0)),
            pl.BlockSpec((1, NCHUNK, CHUNK), lambda i: (i, 0, 0)),
        ],
        out_specs=pl.BlockSpec((NSEG, D), lambda i: (0, 0)),
        out_shape=jax.ShapeDtypeStruct((NSEG, D), jnp.float32),
        scratch_shapes=[
            pltpu.VMEM((NSEG, D), jnp.float32),
            pltpu.VMEM((NSEG, 1), jnp.float32),
            pltpu.VMEM((NSEG, 1), jnp.float32),
        ],
        compiler_params=pltpu.CompilerParams(
            dimension_semantics=("arbitrary",)),
    )(x, W1, b1r, w2r, b2r, seg)
    return out
